# baseline (speedup 1.0000x reference)
import functools

import jax
import jax.numpy as jnp
from jax import lax
from jax.experimental import pallas as pl
from jax.experimental.pallas import tpu as pltpu
from jax.experimental.pallas import tpu_sc as plsc

NC = 2
NS = 16
NW = NC * NS
CHUNK = 32
NBUF = 3


def _sc_body(ids_hbm, table_hbm, out_hbm, idx_v,
             rows0, rows1, rows2, sin0, sin1, sin2, sout0, sout1, sout2):
    wid = lax.axis_index("s") * NC + lax.axis_index("c")
    n_tok = ids_hbm.shape[0]
    b_per_w = n_tok // NW
    base = wid * b_per_w
    pltpu.sync_copy(ids_hbm.at[pl.ds(base, b_per_w)], idx_v)

    bufs = (rows0, rows1, rows2)
    sems_in = (sin0, sin1, sin2)
    sems_out = (sout0, sout1, sout2)
    nch = b_per_w // CHUNK

    def gather(g):
        return pltpu.async_copy(
            table_hbm.at[idx_v.at[pl.ds(g * CHUNK, CHUNK)]],
            bufs[g % NBUF], sems_in[g % NBUF])

    def scatter(g):
        return pltpu.async_copy(
            bufs[g % NBUF],
            out_hbm.at[pl.ds(base + g * CHUNK, CHUNK)],
            sems_out[g % NBUF])

    d_in = [None] * nch
    d_out = [None] * nch
    for g in range(min(NBUF, nch)):
        d_in[g] = gather(g)
    for g in range(nch):
        d_in[g].wait()
        d_out[g] = scatter(g)
        if g + NBUF < nch:
            d_out[g].wait()
            d_in[g + NBUF] = gather(g + NBUF)
    for g in range(max(0, nch - NBUF), nch):
        d_out[g].wait()


def kernel(token_type_ids, W_token_type):
    B, S = token_type_ids.shape
    D = W_token_type.shape[1]
    n_tok = B * S
    ids = token_type_ids.reshape(n_tok).astype(jnp.int32)
    mesh = plsc.VectorSubcoreMesh(
        core_axis_name="c", subcore_axis_name="s",
        num_cores=NC, num_subcores=NS,
    )
    out = pl.kernel(
        _sc_body,
        out_type=jax.ShapeDtypeStruct((n_tok, D), jnp.float32),
        mesh=mesh,
        scratch_types=(
            [pltpu.VMEM((n_tok // NW,), jnp.int32)]
            + [pltpu.VMEM((CHUNK, D), jnp.float32) for _ in range(NBUF)]
            + [pltpu.SemaphoreType.DMA for _ in range(2 * NBUF)]
        ),
    )(ids, W_token_type)
    return out.reshape(B, S, D)


# speedup vs baseline: 2.2738x; 2.2738x over previous
import functools

import jax
import jax.numpy as jnp
from jax import lax
from jax.experimental import pallas as pl
from jax.experimental.pallas import tpu as pltpu
from jax.experimental.pallas import tpu_sc as plsc

NC = 2
NS = 16
NW = NC * NS
L = 16
CHUNK = 32
NBUF = 2


def _sc_body(ids_hbm, table_hbm, out_hbm, idx_v, table_v, buf0, buf1,
             sem0, sem1):
    D = table_hbm.shape[0] // 2
    wid = lax.axis_index("s") * NC + lax.axis_index("c")
    n_tok = ids_hbm.shape[0]
    b_per_w = n_tok // NW
    base = wid * b_per_w

    pltpu.sync_copy(table_hbm, table_v)
    pltpu.sync_copy(ids_hbm.at[pl.ds(base, b_per_w)], idx_v)

    bufs = (buf0, buf1)
    sems = (sem0, sem1)
    lanes = lax.iota(jnp.int32, L)

    def construct(buf, g):
        def tok_body(t, carry):
            tok = g * CHUNK + t
            id_vec = plsc.load_gather(idx_v, [jnp.full((L,), tok, jnp.int32)])
            addr = id_vec * D + lanes
            for j in range(D // L):
                vals = plsc.load_gather(table_v, [addr + (j * L)])
                buf[pl.ds(t * D + j * L, L)] = vals
            return carry
        lax.fori_loop(0, CHUNK, tok_body, 0)

    ngroups = b_per_w // (CHUNK * NBUF)

    def group(go, carry):
        for b in range(NBUF):
            g = go * NBUF + b

            @pl.when(go > 0)
            def _():
                pltpu.make_async_copy(
                    bufs[b], out_hbm.at[pl.ds(base * D, CHUNK * D)], sems[b]
                ).wait()

            construct(bufs[b], g)
            pltpu.async_copy(
                bufs[b],
                out_hbm.at[pl.ds((base + g * CHUNK) * D, CHUNK * D)],
                sems[b])
        return carry

    lax.fori_loop(0, ngroups, group, 0)
    for b in range(NBUF):
        pltpu.make_async_copy(
            bufs[b], out_hbm.at[pl.ds(base * D, CHUNK * D)], sems[b]
        ).wait()


def kernel(token_type_ids, W_token_type):
    B, S = token_type_ids.shape
    D = W_token_type.shape[1]
    n_tok = B * S
    ids = token_type_ids.reshape(n_tok).astype(jnp.int32)
    table = W_token_type.reshape(2 * D)
    mesh = plsc.VectorSubcoreMesh(
        core_axis_name="c", subcore_axis_name="s",
        num_cores=NC, num_subcores=NS,
    )
    out = pl.kernel(
        _sc_body,
        out_type=jax.ShapeDtypeStruct((n_tok * D,), jnp.float32),
        mesh=mesh,
        compiler_params=pltpu.CompilerParams(needs_layout_passes=False),
        scratch_types=[
            pltpu.VMEM((n_tok // NW,), jnp.int32),
            pltpu.VMEM((2 * D,), jnp.float32),
            pltpu.VMEM((CHUNK * D,), jnp.float32),
            pltpu.VMEM((CHUNK * D,), jnp.float32),
            pltpu.SemaphoreType.DMA,
            pltpu.SemaphoreType.DMA,
        ],
    )(ids, table)
    return out.reshape(B, S, D)


# speedup vs baseline: 3.7931x; 1.6682x over previous
import functools

import jax
import jax.numpy as jnp
from jax import lax
from jax.experimental import pallas as pl
from jax.experimental.pallas import tpu as pltpu
from jax.experimental.pallas import tpu_sc as plsc

NC = 2
NS = 16
NW = NC * NS
L = 16
CHUNK = 32
NBUF = 2


def _sc_body(ids_hbm, table_hbm, out_hbm, idx_v, table_v, buf0, buf1,
             sem0, sem1):
    D = table_hbm.shape[0] // 2
    wid = lax.axis_index("s") * NC + lax.axis_index("c")
    n_tok = ids_hbm.shape[0]
    b_per_w = n_tok // NW
    base = wid * b_per_w

    pltpu.sync_copy(table_hbm, table_v)
    pltpu.sync_copy(ids_hbm.at[pl.ds(base, b_per_w)], idx_v)

    bufs = (buf0, buf1)
    sems = (sem0, sem1)
    lanes = lax.iota(jnp.int32, L)

    def construct(buf, g):
        BJ = 8

        NB = D // L // BJ

        def tok_body(t, carry):
            tok = g * CHUNK + t
            id_vec = plsc.load_gather(idx_v, [jnp.full((L,), tok, jnp.int32)])
            addr = id_vec * D + lanes
            tbase = t * D

            def gathers(jb):
                return [
                    plsc.load_gather(
                        table_v.at[pl.ds((jb * BJ + k) * L, D + L)], [addr])
                    for k in range(BJ)
                ]

            def stores(jb, vs):
                for k in range(BJ):
                    buf[pl.ds(tbase + (jb * BJ + k) * L, L)] = vs[k]

            prev = gathers(0)
            for jb in range(1, NB):
                cur = gathers(jb)
                stores(jb - 1, prev)
                prev = cur
            stores(NB - 1, prev)
            return carry
        lax.fori_loop(0, CHUNK, tok_body, 0)

    ngroups = b_per_w // (CHUNK * NBUF)

    def group(go, carry):
        for b in range(NBUF):
            g = go * NBUF + b

            @pl.when(go > 0)
            def _():
                pltpu.make_async_copy(
                    bufs[b], out_hbm.at[pl.ds(base * D, CHUNK * D)], sems[b]
                ).wait()

            construct(bufs[b], g)
            pltpu.async_copy(
                bufs[b],
                out_hbm.at[pl.ds((base + g * CHUNK) * D, CHUNK * D)],
                sems[b])
        return carry

    lax.fori_loop(0, ngroups, group, 0)
    for b in range(NBUF):
        pltpu.make_async_copy(
            bufs[b], out_hbm.at[pl.ds(base * D, CHUNK * D)], sems[b]
        ).wait()


def kernel(token_type_ids, W_token_type):
    B, S = token_type_ids.shape
    D = W_token_type.shape[1]
    n_tok = B * S
    ids = token_type_ids.reshape(n_tok).astype(jnp.int32)
    table = W_token_type.reshape(2 * D)
    mesh = plsc.VectorSubcoreMesh(
        core_axis_name="c", subcore_axis_name="s",
        num_cores=NC, num_subcores=NS,
    )
    out = pl.kernel(
        _sc_body,
        out_type=jax.ShapeDtypeStruct((n_tok * D,), jnp.float32),
        mesh=mesh,
        compiler_params=pltpu.CompilerParams(needs_layout_passes=False),
        scratch_types=[
            pltpu.VMEM((n_tok // NW,), jnp.int32),
            pltpu.VMEM((2 * D,), jnp.float32),
            pltpu.VMEM((CHUNK * D,), jnp.float32),
            pltpu.VMEM((CHUNK * D,), jnp.float32),
            pltpu.SemaphoreType.DMA,
            pltpu.SemaphoreType.DMA,
        ],
    )(ids, table)
    return out.reshape(B, S, D)


# speedup vs baseline: 13.6054x; 3.5869x over previous
import jax
import jax.numpy as jnp
from jax import lax
from jax.experimental import pallas as pl
from jax.experimental.pallas import tpu as pltpu
from jax.experimental.pallas import tpu_sc as plsc

NC = 2
NS = 16
NW = NC * NS
L = 16


def _sc_body(ids_hbm, table_hbm, out_hbm, idx_v, table_v, sem):
    wid = lax.axis_index("s") * NC + lax.axis_index("c")
    n_tok = ids_hbm.shape[0]
    b_per_w = n_tok // NW
    base = wid * b_per_w

    pltpu.sync_copy(table_hbm, table_v)
    pltpu.sync_copy(ids_hbm.at[pl.ds(base, b_per_w)], idx_v)

    def tok_body(t, carry):
        id_vec = plsc.load_gather(idx_v, [jnp.full((L,), t, jnp.int32)])
        id_s = lax.reduce_max(id_vec, axes=(0,))
        pltpu.async_copy(
            table_v.at[pl.ds(id_s, 1)],
            out_hbm.at[pl.ds(base + t, 1)],
            sem)
        return carry

    lax.fori_loop(0, b_per_w, tok_body, 0)

    def drain_body(t, carry):
        pltpu.make_async_copy(
            table_v.at[pl.ds(0, 1)], out_hbm.at[pl.ds(base, 1)], sem
        ).wait()
        return carry

    lax.fori_loop(0, b_per_w, drain_body, 0)


def kernel(token_type_ids, W_token_type):
    B, S = token_type_ids.shape
    D = W_token_type.shape[1]
    n_tok = B * S
    ids = token_type_ids.reshape(n_tok).astype(jnp.int32)
    mesh = plsc.VectorSubcoreMesh(
        core_axis_name="c", subcore_axis_name="s",
        num_cores=NC, num_subcores=NS,
    )
    out = pl.kernel(
        _sc_body,
        out_type=jax.ShapeDtypeStruct((n_tok, D), jnp.float32),
        mesh=mesh,
        compiler_params=pltpu.CompilerParams(needs_layout_passes=False),
        scratch_types=[
            pltpu.VMEM((n_tok // NW,), jnp.int32),
            pltpu.VMEM((2, D), jnp.float32),
            pltpu.SemaphoreType.DMA,
        ],
    )(ids, W_token_type)
    return out.reshape(B, S, D)
